# baseline (speedup 1.0000x reference)
import functools

import jax
import jax.numpy as jnp
from jax import lax
from jax.experimental import pallas as pl
from jax.experimental.pallas import tpu as pltpu
from jax.experimental.pallas import tpu_sc as plsc

_NW = 32
_SPAN = 125_000
_CH = 24_960
_NCH = 5
_TAIL = 200

_mesh = plsc.VectorSubcoreMesh(core_axis_name="c", subcore_axis_name="s")


@functools.partial(
    pl.kernel,
    mesh=_mesh,
    out_type=jax.ShapeDtypeStruct((_NW, 16), jnp.float32),
    scratch_types=[
        pltpu.VMEM((_CH,), jnp.float32),
        pltpu.VMEM((_CH,), jnp.float32),
        pltpu.VMEM((_TAIL,), jnp.float32),
        pltpu.VMEM((16,), jnp.float32),
        pltpu.SemaphoreType.DMA,
        pltpu.SemaphoreType.DMA,
        pltpu.SemaphoreType.DMA,
    ],
)
def _sc_reduce(x_hbm, out_hbm, buf0, buf1, tail_v, acc_v, sem0, sem1, sem2):
    wid = lax.axis_index("s") * 2 + lax.axis_index("c")
    half = wid // 16
    base = (10_000_000 + half * 10_000_000) + (wid % 16) * _SPAN
    sems = (sem0, sem1)

    bufs = (buf0, buf1)

    def _start(idx):
        return pltpu.async_copy(
            x_hbm.at[pl.ds(base + idx * _CH, _CH)],
            bufs[idx % 2],
            sems[idx % 2],
        )

    cp = {0: _start(0), 1: _start(1)}
    tail_cp = pltpu.async_copy(
        x_hbm.at[pl.ds(base + _NCH * _CH, _TAIL)], tail_v, sem2
    )

    zeros = jnp.zeros((16,), jnp.float32)
    accs = (zeros, zeros, zeros, zeros)

    for c in range(_NCH):
        cp[c].wait()
        bf = bufs[c % 2]

        def body(i, a, _bf=bf):
            a0, a1, a2, a3 = a
            off = i * 64
            a0 = jnp.maximum(a0, jnp.abs(_bf[pl.ds(off, 16)]))
            a1 = jnp.maximum(a1, jnp.abs(_bf[pl.ds(off + 16, 16)]))
            a2 = jnp.maximum(a2, jnp.abs(_bf[pl.ds(off + 32, 16)]))
            a3 = jnp.maximum(a3, jnp.abs(_bf[pl.ds(off + 48, 16)]))
            return (a0, a1, a2, a3)

        accs = lax.fori_loop(0, _CH // 64, body, accs)
        if c + 2 < _NCH:
            cp[c + 2] = _start(c + 2)

    tail_cp.wait()
    a0, a1, a2, a3 = accs
    for off in (0, 16, 32, 48, 64, 80, 96, 112, 128, 144, 160, 176, 184):
        a0 = jnp.maximum(a0, jnp.abs(tail_v[pl.ds(off, 16)]))
    acc = jnp.maximum(jnp.maximum(a0, a1), jnp.maximum(a2, a3))
    acc_v[...] = acc
    pltpu.sync_copy(acc_v, out_hbm.at[wid])


def kernel(x):
    xf = x.reshape(-1)
    partial = _sc_reduce(xf)
    return jnp.any(partial != 0.0).reshape(1)


# speedup vs baseline: 184.4439x; 184.4439x over previous
import jax
import jax.numpy as jnp
from jax.experimental import pallas as pl
from jax.experimental.pallas import tpu as pltpu

_CH = 400_000
_NCH = 5


def _body(x_hbm, out_ref, buf, sems):
    t = pl.program_id(0)

    def _cp(idx):
        return pltpu.make_async_copy(
            x_hbm.at[0, pl.ds(1, 2), pl.ds(idx * _CH, _CH)],
            buf.at[idx % 3],
            sems.at[idx % 3],
        )

    @pl.when(t == 0)
    def _init():
        out_ref[0, 0] = 0
        for k in range(2):
            _cp(k).start()

    @pl.when(t + 2 < _NCH)
    def _prefetch():
        _cp(t + 2).start()

    _cp(t).wait()
    nz = jnp.any(buf[t % 3] != 0.0).astype(jnp.int32)
    out_ref[0, 0] = out_ref[0, 0] | nz


def kernel(x):
    xt = jnp.swapaxes(x, 0, 1)
    res = pl.pallas_call(
        _body,
        grid=(_NCH,),
        in_specs=[pl.BlockSpec(memory_space=pl.ANY)],
        out_specs=pl.BlockSpec(memory_space=pltpu.SMEM),
        out_shape=jax.ShapeDtypeStruct((1, 1), jnp.int32),
        compiler_params=pltpu.CompilerParams(vmem_limit_bytes=100 * 1024 * 1024),
        scratch_shapes=[
            pltpu.VMEM((3, 2, _CH), jnp.float32),
            pltpu.SemaphoreType.DMA((3,)),
        ],
    )(xt)
    return (res[0, 0] != 0).reshape(1)
